# baseline (speedup 1.0000x reference)
import jax
import jax.numpy as jnp
import numpy as np
from jax.experimental import pallas as pl
from jax.experimental.pallas import tpu as pltpu

_N_CLS = (5, 3, 4, 6, 10, 19, 4)
_E_DIM = (16, 8, 8, 16, 24, 32, 8)
_ROW_OFF = (0, 5, 8, 12, 18, 28, 47)
_COL_OFF = (0, 16, 24, 32, 48, 72, 104)
_NUM_ROW = 51
_ONE_ROW = 52
_K = 64
_D_IN = 128
_D_MODEL = 256
_L = 50
_BB = 256


def _sel_consts():
    s = np.zeros((8, _K), np.float32)
    base = np.full((1, _K), -1000.0, np.float32)
    for f in range(7):
        lo, hi = _ROW_OFF[f], _ROW_OFF[f] + _N_CLS[f]
        s[f, lo:hi] = 1.0
        base[0, lo:hi] = _ROW_OFF[f]
    base[0, _ONE_ROW] = _ONE_ROW
    return jnp.asarray(s), jnp.asarray(base)


def _fused_kernel(x_ref, t_ref, w_ref, pb_ref, s_ref, base_ref,
                  g_ref, b_ref, out_ref, a_ref):
    @pl.when(pl.program_id(0) == 0)
    def _():
        a = jnp.dot(t_ref[...], w_ref[...], preferred_element_type=jnp.float32)
        row = jax.lax.broadcasted_iota(jnp.int32, (_K, 1), 0)
        a_ref[...] = a + jnp.where(row == _ONE_ROW, 1.0, 0.0) * pb_ref[...]

    lane = jax.lax.broadcasted_iota(
        jnp.int32, (_BB, _K), 1).astype(jnp.float32)
    ones_col = jnp.full((_D_MODEL, 1), 1.0 / _D_MODEL, jnp.float32)
    for l in range(_L):
        x = x_ref[:, 8 * l:8 * l + 8]
        s = jnp.dot(x, s_ref[...],
                    preferred_element_type=jnp.float32) + base_ref[...]
        m = (s == lane).astype(jnp.float32)
        h = jnp.dot(m, a_ref[...], preferred_element_type=jnp.float32)
        h += jnp.dot(x[:, 7:8], a_ref[_NUM_ROW:_NUM_ROW + 1, :],
                     preferred_element_type=jnp.float32)
        h = h * (0.5 * jax.lax.erf(h * 0.7071067811865476) + 0.5)
        mu = jnp.dot(h, ones_col, preferred_element_type=jnp.float32)
        q = jnp.dot(h * h, ones_col, preferred_element_type=jnp.float32)
        k = jax.lax.rsqrt(q - mu * mu + 1e-5)
        kg = jnp.dot(k, g_ref[...], preferred_element_type=jnp.float32)
        off = b_ref[...] - jnp.dot(mu * k, g_ref[...],
                                   preferred_element_type=jnp.float32)
        out_ref[l] = h * kg + off


@jax.jit
def _run(cat_seq, num_seq, tables, num_W, num_b, proj_W, proj_b, ln_g, ln_b):
    B, L, _ = cat_seq.shape

    x8 = jnp.concatenate([cat_seq.astype(jnp.float32), num_seq], axis=2)
    x8 = x8.reshape(B, L * 8)

    t = jnp.zeros((_K, _D_IN), jnp.float32)
    for i in range(7):
        t = jax.lax.dynamic_update_slice(
            t, tables[i], (_ROW_OFF[i], _COL_OFF[i]))
    t = jax.lax.dynamic_update_slice(t, num_W.reshape(1, 16), (_NUM_ROW, 112))
    t = jax.lax.dynamic_update_slice(t, num_b.reshape(1, 16), (_ONE_ROW, 112))
    s_mat, base = _sel_consts()

    grid = (B // _BB,)
    out = pl.pallas_call(
        _fused_kernel,
        grid=grid,
        in_specs=[
            pl.BlockSpec((_BB, L * 8), lambda i: (i, 0)),
            pl.BlockSpec((_K, _D_IN), lambda i: (0, 0)),
            pl.BlockSpec((_D_IN, _D_MODEL), lambda i: (0, 0)),
            pl.BlockSpec((1, _D_MODEL), lambda i: (0, 0)),
            pl.BlockSpec((8, _K), lambda i: (0, 0)),
            pl.BlockSpec((1, _K), lambda i: (0, 0)),
            pl.BlockSpec((1, _D_MODEL), lambda i: (0, 0)),
            pl.BlockSpec((1, _D_MODEL), lambda i: (0, 0)),
        ],
        out_specs=pl.BlockSpec((L, _BB, _D_MODEL), lambda i: (0, i, 0)),
        out_shape=jax.ShapeDtypeStruct((L, B, _D_MODEL), jnp.float32),
        scratch_shapes=[pltpu.VMEM((_K, _D_MODEL), jnp.float32)],
        compiler_params=pltpu.CompilerParams(
            dimension_semantics=("arbitrary",)),
    )(x8, t, proj_W, proj_b.reshape(1, -1), s_mat, base,
      ln_g.reshape(1, -1), ln_b.reshape(1, -1))
    return jnp.transpose(out, (1, 0, 2))


def kernel(cat_seq, num_seq, emb_strikeId, emb_handId, emb_strengthId,
           emb_spinId, emb_pointId, emb_actionId, emb_positionId,
           num_W, num_b, proj_W, proj_b, ln_g, ln_b):
    tables = (emb_strikeId, emb_handId, emb_strengthId, emb_spinId,
              emb_pointId, emb_actionId, emb_positionId)
    return _run(cat_seq, num_seq, tables, num_W, num_b, proj_W, proj_b,
                ln_g, ln_b)


# speedup vs baseline: 1.5975x; 1.5975x over previous
import jax
import jax.numpy as jnp
import numpy as np
from jax.experimental import pallas as pl
from jax.experimental.pallas import tpu as pltpu

_N_CLS = (5, 3, 4, 6, 10, 19, 4)
_E_DIM = (16, 8, 8, 16, 24, 32, 8)
_ROW_OFF = (0, 5, 8, 12, 18, 28, 47)
_COL_OFF = (0, 16, 24, 32, 48, 72, 104)
_NUM_ROW = 51
_ONE_ROW = 52
_K = 64
_D_IN = 128
_D_MODEL = 256
_L = 50
_BB = 512


def _sel_consts():
    s = np.zeros((8, _K), np.float32)
    base = np.full((1, _K), -1000.0, np.float32)
    for f in range(7):
        lo, hi = _ROW_OFF[f], _ROW_OFF[f] + _N_CLS[f]
        s[f, lo:hi] = 1.0
        base[0, lo:hi] = _ROW_OFF[f]
    base[0, _ONE_ROW] = _ONE_ROW
    return jnp.asarray(s), jnp.asarray(base)


def _fused_kernel(x_ref, t_ref, w_ref, pb_ref, s_ref, base_ref,
                  g_ref, b_ref, out_ref, a_ref):
    @pl.when(pl.program_id(0) == 0)
    def _():
        a = jnp.dot(t_ref[...], w_ref[...], preferred_element_type=jnp.float32)
        row = jax.lax.broadcasted_iota(jnp.int32, (_K, 1), 0)
        a_ref[...] = a + jnp.where(row == _ONE_ROW, 1.0, 0.0) * pb_ref[...]

    lane = jax.lax.broadcasted_iota(
        jnp.int32, (_BB, _K), 1).astype(jnp.float32)
    for l in range(_L):
        x = x_ref[:, 8 * l:8 * l + 8]
        s = jnp.dot(x, s_ref[...],
                    preferred_element_type=jnp.float32) + base_ref[...]
        m = (s == lane).astype(jnp.float32)
        h = jnp.dot(m, a_ref[...], preferred_element_type=jnp.float32)
        h += jnp.dot(x[:, 7:8], a_ref[_NUM_ROW:_NUM_ROW + 1, :],
                     preferred_element_type=jnp.float32)
        h = h * (0.5 * jax.lax.erf(h * 0.7071067811865476) + 0.5)
        mu = jnp.mean(h, axis=1, keepdims=True)
        d = h - mu
        var = jnp.mean(d * d, axis=1, keepdims=True)
        out_ref[l] = d * jax.lax.rsqrt(var + 1e-5) * g_ref[...] + b_ref[...]


@jax.jit
def _run(cat_seq, num_seq, tables, num_W, num_b, proj_W, proj_b, ln_g, ln_b):
    B, L, _ = cat_seq.shape

    x8 = jnp.concatenate([cat_seq.astype(jnp.float32), num_seq], axis=2)
    x8 = x8.reshape(B, L * 8)

    t = jnp.zeros((_K, _D_IN), jnp.float32)
    for i in range(7):
        t = jax.lax.dynamic_update_slice(
            t, tables[i], (_ROW_OFF[i], _COL_OFF[i]))
    t = jax.lax.dynamic_update_slice(t, num_W.reshape(1, 16), (_NUM_ROW, 112))
    t = jax.lax.dynamic_update_slice(t, num_b.reshape(1, 16), (_ONE_ROW, 112))
    s_mat, base = _sel_consts()

    grid = (B // _BB,)
    out = pl.pallas_call(
        _fused_kernel,
        grid=grid,
        in_specs=[
            pl.BlockSpec((_BB, L * 8), lambda i: (i, 0)),
            pl.BlockSpec((_K, _D_IN), lambda i: (0, 0)),
            pl.BlockSpec((_D_IN, _D_MODEL), lambda i: (0, 0)),
            pl.BlockSpec((1, _D_MODEL), lambda i: (0, 0)),
            pl.BlockSpec((8, _K), lambda i: (0, 0)),
            pl.BlockSpec((1, _K), lambda i: (0, 0)),
            pl.BlockSpec((1, _D_MODEL), lambda i: (0, 0)),
            pl.BlockSpec((1, _D_MODEL), lambda i: (0, 0)),
        ],
        out_specs=pl.BlockSpec((L, _BB, _D_MODEL), lambda i: (0, i, 0)),
        out_shape=jax.ShapeDtypeStruct((L, B, _D_MODEL), jnp.float32),
        scratch_shapes=[pltpu.VMEM((_K, _D_MODEL), jnp.float32)],
        compiler_params=pltpu.CompilerParams(
            dimension_semantics=("arbitrary",)),
    )(x8, t, proj_W, proj_b.reshape(1, -1), s_mat, base,
      ln_g.reshape(1, -1), ln_b.reshape(1, -1))
    return jnp.transpose(out, (1, 0, 2))


def kernel(cat_seq, num_seq, emb_strikeId, emb_handId, emb_strengthId,
           emb_spinId, emb_pointId, emb_actionId, emb_positionId,
           num_W, num_b, proj_W, proj_b, ln_g, ln_b):
    tables = (emb_strikeId, emb_handId, emb_strengthId, emb_spinId,
              emb_pointId, emb_actionId, emb_positionId)
    return _run(cat_seq, num_seq, tables, num_W, num_b, proj_W, proj_b,
                ln_g, ln_b)


# speedup vs baseline: 1.8860x; 1.1806x over previous
import jax
import jax.numpy as jnp
import numpy as np
from jax.experimental import pallas as pl
from jax.experimental.pallas import tpu as pltpu

_N_CLS = (5, 3, 4, 6, 10, 19, 4)
_E_DIM = (16, 8, 8, 16, 24, 32, 8)
_ROW_OFF = (0, 5, 8, 12, 18, 28, 47)
_COL_OFF = (0, 16, 24, 32, 48, 72, 104)
_ONE_ROW = 52
_NUM_LANE = 53
_K = 64
_D_IN = 128
_D_MODEL = 256
_L = 50
_BB = 512


def _sel_consts():
    s = np.zeros((8, _K), np.float32)
    base = np.full((1, _K), -1000.0, np.float32)
    for f in range(7):
        lo, hi = _ROW_OFF[f], _ROW_OFF[f] + _N_CLS[f]
        s[f, lo:hi] = 1.0
        base[0, lo:hi] = _ROW_OFF[f]
    base[0, _ONE_ROW] = _ONE_ROW
    s[7, _NUM_LANE] = 1.0
    base[0, _NUM_LANE] = 0.0
    return jnp.asarray(s), jnp.asarray(base)


def _fused_kernel(x_ref, t_ref, w_ref, pb_ref, s_ref, base_ref,
                  g_ref, b_ref, out_ref, a_ref):
    @pl.when(pl.program_id(0) == 0)
    def _():
        a = jnp.dot(t_ref[...], w_ref[...], preferred_element_type=jnp.float32)
        row = jax.lax.broadcasted_iota(jnp.int32, (_K, 1), 0)
        a_ref[...] = a + jnp.where(row == _ONE_ROW, 1.0, 0.0) * pb_ref[...]

    lane = jax.lax.broadcasted_iota(
        jnp.int32, (_BB, _K), 1).astype(jnp.float32)
    for l in range(_L):
        x = x_ref[:, 8 * l:8 * l + 8]
        s = jnp.dot(x, s_ref[...],
                    preferred_element_type=jnp.float32) + base_ref[...]
        m = (s == lane).astype(jnp.float32)
        m = jnp.where(lane == _NUM_LANE, s, m)
        h = jnp.dot(m, a_ref[...], preferred_element_type=jnp.float32)
        h = h * (0.5 * jax.lax.erf(h * 0.7071067811865476) + 0.5)
        mu = jnp.mean(h, axis=1, keepdims=True)
        d = h - mu
        var = jnp.mean(d * d, axis=1, keepdims=True)
        out_ref[l] = d * jax.lax.rsqrt(var + 1e-5) * g_ref[...] + b_ref[...]


@jax.jit
def _run(cat_seq, num_seq, tables, num_W, num_b, proj_W, proj_b, ln_g, ln_b):
    B, L, _ = cat_seq.shape

    x8 = jnp.concatenate([cat_seq.astype(jnp.float32), num_seq], axis=2)
    x8 = x8.reshape(B, L * 8)

    t = jnp.zeros((_K, _D_IN), jnp.float32)
    for i in range(7):
        t = jax.lax.dynamic_update_slice(
            t, tables[i], (_ROW_OFF[i], _COL_OFF[i]))
    t = jax.lax.dynamic_update_slice(t, num_W.reshape(1, 16), (_NUM_LANE, 112))
    t = jax.lax.dynamic_update_slice(t, num_b.reshape(1, 16), (_ONE_ROW, 112))
    s_mat, base = _sel_consts()

    grid = (B // _BB,)
    out = pl.pallas_call(
        _fused_kernel,
        grid=grid,
        in_specs=[
            pl.BlockSpec((_BB, L * 8), lambda i: (i, 0)),
            pl.BlockSpec((_K, _D_IN), lambda i: (0, 0)),
            pl.BlockSpec((_D_IN, _D_MODEL), lambda i: (0, 0)),
            pl.BlockSpec((1, _D_MODEL), lambda i: (0, 0)),
            pl.BlockSpec((8, _K), lambda i: (0, 0)),
            pl.BlockSpec((1, _K), lambda i: (0, 0)),
            pl.BlockSpec((1, _D_MODEL), lambda i: (0, 0)),
            pl.BlockSpec((1, _D_MODEL), lambda i: (0, 0)),
        ],
        out_specs=pl.BlockSpec((L, _BB, _D_MODEL), lambda i: (0, i, 0)),
        out_shape=jax.ShapeDtypeStruct((L, B, _D_MODEL), jnp.float32),
        scratch_shapes=[pltpu.VMEM((_K, _D_MODEL), jnp.float32)],
        compiler_params=pltpu.CompilerParams(
            dimension_semantics=("arbitrary",)),
    )(x8, t, proj_W, proj_b.reshape(1, -1), s_mat, base,
      ln_g.reshape(1, -1), ln_b.reshape(1, -1))
    return jnp.transpose(out, (1, 0, 2))


def kernel(cat_seq, num_seq, emb_strikeId, emb_handId, emb_strengthId,
           emb_spinId, emb_pointId, emb_actionId, emb_positionId,
           num_W, num_b, proj_W, proj_b, ln_g, ln_b):
    tables = (emb_strikeId, emb_handId, emb_strengthId, emb_spinId,
              emb_pointId, emb_actionId, emb_positionId)
    return _run(cat_seq, num_seq, tables, num_W, num_b, proj_W, proj_b,
                ln_g, ln_b)
